# Initial kernel scaffold; baseline (speedup 1.0000x reference)
import functools

import jax
import jax.numpy as jnp
from jax import lax
from jax.experimental import pallas as pl
from jax.experimental.pallas import tpu as pltpu
from jax.experimental.pallas import tpu_sc as plsc

_INFO = plsc.get_sparse_core_info()
_NW = _INFO.num_cores * _INFO.num_subcores

_CHUNK = 1024


@functools.lru_cache(maxsize=None)
def _make_gather(B, V, D):
    b_per_w = B // _NW
    n_chunks = b_per_w // _CHUNK
    mesh = plsc.VectorSubcoreMesh(core_axis_name="c", subcore_axis_name="s")

    @functools.partial(
        pl.kernel,
        out_type=jax.ShapeDtypeStruct((B, D), jnp.float32),
        mesh=mesh,
        scratch_types=[
            pltpu.VMEM((b_per_w,), jnp.int32),
            pltpu.VMEM((_CHUNK, D), jnp.float32),
            pltpu.VMEM((_CHUNK, D), jnp.float32),
            pltpu.SemaphoreType.DMA,
            pltpu.SemaphoreType.DMA,
        ],
    )
    def gather_kernel(idx_hbm, table_hbm, out_hbm, idx_v, rows0, rows1, gsem, wsem):
        wid = lax.axis_index("s") * _INFO.num_cores + lax.axis_index("c")
        base = wid * b_per_w
        pltpu.sync_copy(idx_hbm.at[pl.ds(base, b_per_w)], idx_v)
        rows = (rows0, rows1)
        pending = [None, None]
        for i in range(n_chunks):
            buf = rows[i % 2]
            if pending[i % 2] is not None:
                pending[i % 2].wait()
            pltpu.async_copy(
                table_hbm.at[idx_v.at[pl.ds(i * _CHUNK, _CHUNK)]], buf, gsem
            ).wait()
            pending[i % 2] = pltpu.async_copy(
                buf, out_hbm.at[pl.ds(base + i * _CHUNK, _CHUNK)], wsem
            )
        for d in pending:
            if d is not None:
                d.wait()

    return gather_kernel


def kernel(input, W_E):
    B = input.shape[0] * input.shape[1]
    V, D = W_E.shape
    idx = input.reshape(-1).astype(jnp.int32)
    grain = _NW * _CHUNK
    Bp = ((B + grain - 1) // grain) * grain
    if Bp != B:
        idx = jnp.pad(idx, (0, Bp - B))
    out = _make_gather(Bp, V, D)(idx, W_E)
    if Bp != B:
        out = out[:B]
    return out.reshape(input.shape[0], input.shape[1], D)


# speedup vs baseline: 1.5666x; 1.5666x over previous
import functools

import jax
import jax.numpy as jnp
from jax import lax
from jax.experimental import pallas as pl
from jax.experimental.pallas import tpu as pltpu
from jax.experimental.pallas import tpu_sc as plsc

_INFO = plsc.get_sparse_core_info()
_NW = _INFO.num_cores * _INFO.num_subcores

_CHUNK = 1024


@functools.lru_cache(maxsize=None)
def _make_gather(B, V, D):
    b_per_w = B // _NW
    n_chunks = b_per_w // _CHUNK
    mesh = plsc.VectorSubcoreMesh(core_axis_name="c", subcore_axis_name="s")

    @functools.partial(
        pl.kernel,
        out_type=jax.ShapeDtypeStruct((B, D), jnp.float32),
        mesh=mesh,
        scratch_types=[
            pltpu.VMEM((b_per_w,), jnp.int32),
            pltpu.VMEM((_CHUNK, D), jnp.float32),
            pltpu.VMEM((_CHUNK, D), jnp.float32),
            pltpu.SemaphoreType.DMA,
            pltpu.SemaphoreType.DMA,
        ],
        compiler_params=pltpu.CompilerParams(use_tc_tiling_on_sc=False),
    )
    def gather_kernel(idx_hbm, table_hbm, out_hbm, idx_v, rows0, rows1, gsem, wsem):
        wid = lax.axis_index("s") * _INFO.num_cores + lax.axis_index("c")
        base = wid * b_per_w
        pltpu.sync_copy(idx_hbm.at[pl.ds(base, b_per_w)], idx_v)
        rows = (rows0, rows1)
        pending = [None, None]
        for i in range(n_chunks):
            buf = rows[i % 2]
            if pending[i % 2] is not None:
                pending[i % 2].wait()
            pltpu.async_copy(
                table_hbm.at[idx_v.at[pl.ds(i * _CHUNK, _CHUNK)]], buf, gsem
            ).wait()
            pending[i % 2] = pltpu.async_copy(
                buf, out_hbm.at[pl.ds(base + i * _CHUNK, _CHUNK)], wsem
            )
        for d in pending:
            if d is not None:
                d.wait()

    return gather_kernel


def kernel(input, W_E):
    B = input.shape[0] * input.shape[1]
    V, D = W_E.shape
    idx = input.reshape(-1).astype(jnp.int32)
    grain = _NW * _CHUNK
    Bp = ((B + grain - 1) // grain) * grain
    if Bp != B:
        idx = jnp.pad(idx, (0, Bp - B))
    out = _make_gather(Bp, V, D)(idx, W_E)
    if Bp != B:
        out = out[:B]
    return out.reshape(input.shape[0], input.shape[1], D)


# speedup vs baseline: 1.5770x; 1.0066x over previous
import functools

import jax
import jax.numpy as jnp
from jax import lax
from jax.experimental import pallas as pl
from jax.experimental.pallas import tpu as pltpu
from jax.experimental.pallas import tpu_sc as plsc

_INFO = plsc.get_sparse_core_info()
_NW = _INFO.num_cores * _INFO.num_subcores

_CHUNK = 1024
_NBUF = 3


@functools.lru_cache(maxsize=None)
def _make_gather(B, V, D):
    b_per_w = B // _NW
    n_chunks = b_per_w // _CHUNK
    mesh = plsc.VectorSubcoreMesh(core_axis_name="c", subcore_axis_name="s")

    @functools.partial(
        pl.kernel,
        out_type=jax.ShapeDtypeStruct((B, D), jnp.float32),
        mesh=mesh,
        scratch_types=[
            pltpu.VMEM((b_per_w,), jnp.int32),
            pltpu.VMEM((_NBUF, _CHUNK, D), jnp.float32),
            pltpu.SemaphoreType.DMA,
            pltpu.SemaphoreType.DMA,
        ],
        compiler_params=pltpu.CompilerParams(use_tc_tiling_on_sc=False),
    )
    def gather_kernel(idx_hbm, table_hbm, out_hbm, idx_v, rows_v, gsem, wsem):
        wid = lax.axis_index("s") * _INFO.num_cores + lax.axis_index("c")
        base = wid * b_per_w
        pltpu.sync_copy(idx_hbm.at[pl.ds(base, b_per_w)], idx_v)

        def gather(i, s):
            return pltpu.async_copy(
                table_hbm.at[idx_v.at[pl.ds(i * _CHUNK, _CHUNK)]],
                rows_v.at[s],
                gsem,
            )

        gd = [None] * _NBUF
        wd = [None] * _NBUF
        for i in range(min(_NBUF, n_chunks)):
            gd[i] = gather(i, i)
        for i in range(n_chunks):
            s = i % _NBUF
            gd[s].wait()
            wd[s] = pltpu.async_copy(
                rows_v.at[s], out_hbm.at[pl.ds(base + i * _CHUNK, _CHUNK)], wsem
            )
            nxt = i + _NBUF
            if nxt < n_chunks:
                wd[s].wait()
                gd[s] = gather(nxt, s)
                wd[s] = None
        for d in wd:
            if d is not None:
                d.wait()

    return gather_kernel


def kernel(input, W_E):
    B = input.shape[0] * input.shape[1]
    V, D = W_E.shape
    idx = input.reshape(-1).astype(jnp.int32)
    grain = _NW * _CHUNK
    Bp = ((B + grain - 1) // grain) * grain
    if Bp != B:
        idx = jnp.pad(idx, (0, Bp - B))
    out = _make_gather(Bp, V, D)(idx, W_E)
    if Bp != B:
        out = out[:B]
    return out.reshape(input.shape[0], input.shape[1], D)


# speedup vs baseline: 1.6314x; 1.0345x over previous
import functools

import jax
import jax.numpy as jnp
from jax import lax
from jax.experimental import pallas as pl
from jax.experimental.pallas import tpu as pltpu
from jax.experimental.pallas import tpu_sc as plsc

_INFO = plsc.get_sparse_core_info()
_NW = _INFO.num_cores * _INFO.num_subcores

_CHUNK = 32
_NBUF = 3


@functools.lru_cache(maxsize=None)
def _make_gather(B, F, V, D):
    rows_per_w = B // _NW
    n_chunks = rows_per_w // _CHUNK
    mesh = plsc.VectorSubcoreMesh(core_axis_name="c", subcore_axis_name="s")

    @functools.partial(
        pl.kernel,
        out_type=jax.ShapeDtypeStruct((B * F, D), jnp.float32),
        mesh=mesh,
        scratch_types=[
            pltpu.VMEM((rows_per_w * F,), jnp.int32),
            pltpu.VMEM((_NBUF, _CHUNK * F, D), jnp.float32),
            pltpu.SemaphoreType.DMA,
            pltpu.SemaphoreType.DMA,
        ],
        compiler_params=pltpu.CompilerParams(use_tc_tiling_on_sc=False),
    )
    def gather_kernel(idx_hbm, table_hbm, out_hbm, idx_v, rows_v, gsem, wsem):
        wid = lax.axis_index("s") * _INFO.num_cores + lax.axis_index("c")
        base = wid * rows_per_w
        out_flat = out_hbm
        pltpu.sync_copy(idx_hbm.at[pl.ds(base * F, rows_per_w * F)], idx_v)

        def gather(i, s):
            return pltpu.async_copy(
                table_hbm.at[idx_v.at[pl.ds(i * _CHUNK * F, _CHUNK * F)]],
                rows_v.at[s],
                gsem,
            )

        gd = [None] * _NBUF
        wd = [None] * _NBUF
        for i in range(min(_NBUF, n_chunks)):
            gd[i] = gather(i, i)
        for i in range(n_chunks):
            s = i % _NBUF
            gd[s].wait()
            wd[s] = pltpu.async_copy(
                rows_v.at[s],
                out_flat.at[pl.ds((base + i * _CHUNK) * F, _CHUNK * F)],
                wsem,
            )
            nxt = i + _NBUF
            if nxt < n_chunks:
                wd[s].wait()
                gd[s] = gather(nxt, s)
                wd[s] = None
        for d in wd:
            if d is not None:
                d.wait()

    return gather_kernel


def kernel(input, W_E):
    B, F = input.shape
    V, D = W_E.shape
    return _make_gather(B, F, V, D)(input.reshape(-1), W_E)
